# baseline (speedup 1.0000x reference)
import jax
import jax.numpy as jnp
import numpy as np
from jax.experimental import pallas as pl
from jax.experimental.pallas import tpu as pltpu
from jax.sharding import Mesh, PartitionSpec as P

_FEA_DIM = 256
_BANK_DIM = 1024
_SHRINK = 0.0025
_BLOCK_M = 1024


def _fused_body(x_ref, bank_ref, o_ref):
    x = x_ref[...].astype(jnp.bfloat16)
    bank = bank_ref[...].astype(jnp.bfloat16)
    a = jax.lax.dot_general(
        x, bank, (((1,), (1,)), ((), ())), preferred_element_type=jnp.float32
    )
    m = jnp.max(a, axis=1, keepdims=True)
    e = jnp.exp(a - m)
    p = e * (1.0 / jnp.sum(e, axis=1, keepdims=True))
    s = jnp.maximum(p - _SHRINK, 0.0)
    e2 = jnp.exp(s)
    inv_z2 = 1.0 / jnp.sum(e2, axis=1, keepdims=True)
    o = jnp.dot(e2.astype(jnp.bfloat16), bank, preferred_element_type=jnp.float32)
    o_ref[...] = jnp.tanh(o * inv_z2)


def _fused_call(x, bank):
    n, f = x.shape
    grid = (n // _BLOCK_M,)
    return pl.pallas_call(
        _fused_body,
        grid=grid,
        in_specs=[
            pl.BlockSpec((_BLOCK_M, f), lambda i: (i, 0)),
            pl.BlockSpec((_BANK_DIM, f), lambda i: (0, 0)),
        ],
        out_specs=pl.BlockSpec((_BLOCK_M, f), lambda i: (i, 0)),
        out_shape=jax.ShapeDtypeStruct((n, f), jnp.float32),
        compiler_params=pltpu.CompilerParams(
            dimension_semantics=("arbitrary",),
        ),
    )(x, bank)


def kernel(input, bank):
    n = input.shape[0]
    devs = jax.devices()
    n_dev = 2 if len(devs) >= 2 and n % (2 * _BLOCK_M) == 0 else 1
    if n_dev == 1:
        return _fused_call(input, bank)
    mesh = Mesh(np.array(devs[:n_dev]), ("d",))
    fn = jax.shard_map(
        _fused_call,
        mesh=mesh,
        in_specs=(P("d", None), P(None, None)),
        out_specs=P("d", None),
        check_vma=False,
    )
    return fn(input, bank)


# speedup vs baseline: 7.5098x; 7.5098x over previous
import jax
import jax.numpy as jnp
from jax.experimental import pallas as pl
from jax.experimental.pallas import tpu as pltpu

_FEA_DIM = 256
_BANK_DIM = 1024
_SHRINK = 0.0025
_BLOCK_M = 2048


def _fused_body(x_ref, bank_ref, o_ref):
    x = x_ref[...].astype(jnp.bfloat16)
    bank = bank_ref[...]
    a = jax.lax.dot_general(
        x, bank, (((1,), (1,)), ((), ())), preferred_element_type=jnp.float32
    )
    m = jnp.max(a, axis=1, keepdims=True)
    e = jnp.exp(a - m)
    p = e * (1.0 / jnp.sum(e, axis=1, keepdims=True))
    s = jnp.maximum(p - _SHRINK, 0.0)
    e2 = jnp.exp(s)
    inv_z2 = 1.0 / jnp.sum(e2, axis=1, keepdims=True)
    o = jnp.dot(e2.astype(jnp.bfloat16), bank, preferred_element_type=jnp.float32)
    o_ref[...] = jnp.tanh(o * inv_z2)


def kernel(input, bank):
    n, f = input.shape
    grid = (n // _BLOCK_M,)
    return pl.pallas_call(
        _fused_body,
        grid=grid,
        in_specs=[
            pl.BlockSpec((_BLOCK_M, f), lambda i: (i, 0)),
            pl.BlockSpec((_BANK_DIM, f), lambda i: (0, 0)),
        ],
        out_specs=pl.BlockSpec((_BLOCK_M, f), lambda i: (i, 0)),
        out_shape=jax.ShapeDtypeStruct((n, f), jnp.float32),
        compiler_params=pltpu.CompilerParams(
            dimension_semantics=("arbitrary",),
        ),
    )(input, bank.astype(jnp.bfloat16))


# speedup vs baseline: 9.8206x; 1.3077x over previous
import jax
import jax.numpy as jnp
from jax.experimental import pallas as pl
from jax.experimental.pallas import tpu as pltpu

_FEA_DIM = 256
_BANK_DIM = 1024
_SHRINK = 0.0025
_BLOCK_M = 2048


_SUB = 8


def _chain(x, bank):
    a = jax.lax.dot_general(
        x, bank, (((1,), (1,)), ((), ())), preferred_element_type=jnp.float32
    )
    m = jnp.max(a, axis=1, keepdims=True)
    e = jnp.exp(a - m)
    p = e * (1.0 / jnp.sum(e, axis=1, keepdims=True))
    s = jnp.maximum(p - _SHRINK, 0.0)
    e2 = jnp.exp(s)
    inv_z2 = 1.0 / jnp.sum(e2, axis=1, keepdims=True)
    o = jnp.dot(e2.astype(jnp.bfloat16), bank, preferred_element_type=jnp.float32)
    return jnp.tanh(o * inv_z2)


def _fused_body(x_ref, bank_ref, o_ref):
    bank = bank_ref[...]
    sub = _BLOCK_M // _SUB
    for k in range(_SUB):
        x = x_ref[k * sub : (k + 1) * sub, :].astype(jnp.bfloat16)
        o_ref[k * sub : (k + 1) * sub, :] = _chain(x, bank)


def kernel(input, bank):
    n, f = input.shape
    grid = (n // _BLOCK_M,)
    return pl.pallas_call(
        _fused_body,
        grid=grid,
        in_specs=[
            pl.BlockSpec((_BLOCK_M, f), lambda i: (i, 0)),
            pl.BlockSpec((_BANK_DIM, f), lambda i: (0, 0)),
        ],
        out_specs=pl.BlockSpec((_BLOCK_M, f), lambda i: (i, 0)),
        out_shape=jax.ShapeDtypeStruct((n, f), jnp.float32),
        compiler_params=pltpu.CompilerParams(
            dimension_semantics=("arbitrary",),
        ),
    )(input, bank.astype(jnp.bfloat16))
